# Initial kernel scaffold; baseline (speedup 1.0000x reference)
import jax
import jax.numpy as jnp
from jax.experimental import pallas as pl


def kernel(snoRNAs, diseases, emb_sno, emb_dis, W_fc, b_fc, W_fcg, b_fcg, graph_rows, graph_cols, graph_vals):
    raise NotImplementedError("write your pallas kernel here")


# speedup vs baseline: 1.4886x; 1.4886x over previous
import functools

import jax
import jax.numpy as jnp
from jax import lax
from jax.experimental import pallas as pl
from jax.experimental.pallas import tpu as pltpu
from jax.experimental.pallas import tpu_sc as plsc

_N_SNO = 50000
_N_DIS = 10000
_T = _N_SNO + _N_DIS
_D = 200
_G = 4
_DP = 256
_B = 4096

_BLK = 2000
_DOT_BLK = 512


def _fc_onehot_body(ego_ref, side_ref, wfc_ref, bfc_ref, wfcg_ref, bfcg_ref,
                    oh_ref):
    i = pl.program_id(0)
    x = ego_ref[...] + side_ref[...]
    y = jnp.dot(x, wfc_ref[...], preferred_element_type=jnp.float32)
    y = y + bfc_ref[...]
    y = jnp.where(y >= 0, y, 0.01 * y)
    s = jnp.dot(y, wfcg_ref[...], preferred_element_type=jnp.float32)
    s = s + bfcg_ref[...]
    top = jnp.max(s, axis=1, keepdims=True)
    oh = (s == top).astype(jnp.float32)
    rows = i * _BLK + lax.broadcasted_iota(jnp.int32, (_BLK, _G), 0)
    oh_ref[...] = jnp.where(rows >= _N_SNO, 1.0, oh)


_fc_onehot = pl.pallas_call(
    _fc_onehot_body,
    grid=(_T // _BLK,),
    in_specs=[
        pl.BlockSpec((_BLK, _D), lambda i: (i, 0)),
        pl.BlockSpec((_BLK, _D), lambda i: (i, 0)),
        pl.BlockSpec((_D, _D), lambda i: (0, 0)),
        pl.BlockSpec((_D,), lambda i: (0,)),
        pl.BlockSpec((_D, _G), lambda i: (0, 0)),
        pl.BlockSpec((_G,), lambda i: (0,)),
    ],
    out_specs=pl.BlockSpec((_BLK, _G), lambda i: (i, 0)),
    out_shape=jax.ShapeDtypeStruct((_T, _G), jnp.float32),
)


def _make_sc_gather(dpad, btot):
    info = plsc.get_sparse_core_info()
    nc, ns = info.num_cores, info.num_subcores
    nw = nc * ns
    b_per_w = btot // nw
    mesh = plsc.VectorSubcoreMesh(core_axis_name="c", subcore_axis_name="s")

    @functools.partial(
        pl.kernel, mesh=mesh,
        out_type=jax.ShapeDtypeStruct((btot, dpad), jnp.float32),
        scratch_types=[
            pltpu.VMEM((b_per_w,), jnp.int32),
            pltpu.VMEM((b_per_w, dpad), jnp.float32),
            pltpu.SemaphoreType.DMA,
        ],
    )
    def k(table_hbm, idx_hbm, out_hbm, idx_v, rows_v, sem):
        wid = lax.axis_index("s") * nc + lax.axis_index("c")
        base = wid * b_per_w
        pltpu.sync_copy(idx_hbm.at[pl.ds(base, b_per_w)], idx_v)
        pltpu.async_copy(table_hbm.at[idx_v], rows_v, sem).wait()
        pltpu.sync_copy(rows_v, out_hbm.at[pl.ds(base, b_per_w)])

    return k


_sc_gather = _make_sc_gather(_DP, 2 * _B)


def _dot_body(s_ref, d_ref, o_ref):
    o_ref[...] = jnp.sum(s_ref[...] * d_ref[...], axis=1)


_pair_dot = pl.pallas_call(
    _dot_body,
    grid=(_B // _DOT_BLK,),
    in_specs=[
        pl.BlockSpec((_DOT_BLK, _DP), lambda i: (i, 0)),
        pl.BlockSpec((_DOT_BLK, _DP), lambda i: (i + _B // _DOT_BLK, 0)),
    ],
    out_specs=pl.BlockSpec((_DOT_BLK,), lambda i: (i,)),
    out_shape=jax.ShapeDtypeStruct((_B,), jnp.float32),
)


def kernel(snoRNAs, diseases, emb_sno, emb_dis, W_fc, b_fc, W_fcg, b_fcg,
           graph_rows, graph_cols, graph_vals):
    all_emb = jnp.concatenate([emb_sno, emb_dis], axis=0)
    side = jax.ops.segment_sum(all_emb[graph_cols] * graph_vals[:, None],
                               graph_rows, num_segments=_T)
    oh = _fc_onehot(all_emb, side, W_fc, b_fc, W_fcg, b_fcg)

    x = (oh[:, :, None] * all_emb[:, None, :]).reshape(_T, _G * _D)
    acc = jnp.zeros((_T, _D), jnp.float32)
    for _ in range(4):
        y = jax.ops.segment_sum(x[graph_cols] * graph_vals[:, None],
                                graph_rows, num_segments=_T)
        ym = y.reshape(_T, _G, _D) * oh[:, :, None]
        acc = acc + ym.sum(axis=1)
        x = ym.reshape(_T, _G * _D)

    all_out = 0.2 * (float(_G) * all_emb + acc)
    tab = jnp.pad(all_out, ((0, 0), (0, _DP - _D)))
    idx = jnp.concatenate([snoRNAs.astype(jnp.int32),
                           _N_SNO + diseases.astype(jnp.int32)])
    rows = _sc_gather(tab, idx)
    return _pair_dot(rows, rows)
